# baseline (speedup 1.0000x reference)
import functools

import jax
import jax.numpy as jnp
from jax import lax
from jax.experimental import pallas as pl
from jax.experimental.pallas import tpu as pltpu
from jax.experimental.pallas import tpu_sc as plsc

N = 10000
E = 320000
D = 128

NC = 2
NS = 16
NW = NC * NS
CH = 128
NCH = 80
EPW = NCH * CH
E_PAD = NW * EPW
PAD = N
N_PAD = 10240
RPS = N_PAD // NS

_f32 = jnp.float32
_i32 = jnp.int32

_MESH = plsc.VectorSubcoreMesh(core_axis_name="c", subcore_axis_name="s")


@functools.partial(
    pl.kernel,
    out_type=(
        jax.ShapeDtypeStruct((NC, N_PAD), _f32),
        jax.ShapeDtypeStruct((NC, N_PAD), _f32),
    ),
    mesh=_MESH,
    scratch_types=[
        pltpu.VMEM((NCH, CH), _i32),
        pltpu.VMEM((NCH, CH), _i32),
        pltpu.VMEM((CH,), _f32),
        pltpu.VMEM((RPS,), _f32),
        pltpu.VMEM_SHARED((N_PAD,), _f32),
        pltpu.VMEM_SHARED((N_PAD,), _f32),
    ],
)
def _deg_kernel(src_hbm, dst_hbm, dout_hbm, din_hbm,
                src_v, dst_v, ones_v, zeros_v, deg_o, deg_i):
    cid = lax.axis_index("c")
    sid = lax.axis_index("s")
    wid = cid * NS + sid

    def _init(i, _):
        ones_v[pl.ds(i * 16, 16)] = jnp.ones((16,), _f32)
        return 0
    lax.fori_loop(0, CH // 16, _init, 0)

    def _zinit(i, _):
        zeros_v[pl.ds(i * 16, 16)] = jnp.zeros((16,), _f32)
        return 0
    lax.fori_loop(0, RPS // 16, _zinit, 0)

    pltpu.sync_copy(zeros_v, deg_o.at[pl.ds(sid * RPS, RPS)])
    pltpu.sync_copy(zeros_v, deg_i.at[pl.ds(sid * RPS, RPS)])
    plsc.subcore_barrier()

    pltpu.sync_copy(src_hbm.at[wid], src_v)
    pltpu.sync_copy(dst_hbm.at[wid], dst_v)

    def _body(j, _):
        pltpu.sync_copy(ones_v, deg_o.at[src_v.at[j]], add=True)
        pltpu.sync_copy(ones_v, deg_i.at[dst_v.at[j]], add=True)
        return 0
    lax.fori_loop(0, NCH, _body, 0)

    plsc.subcore_barrier()
    pltpu.sync_copy(deg_o.at[pl.ds(sid * RPS, RPS)],
                    dout_hbm.at[cid, pl.ds(sid * RPS, RPS)])
    pltpu.sync_copy(deg_i.at[pl.ds(sid * RPS, RPS)],
                    din_hbm.at[cid, pl.ds(sid * RPS, RPS)])


CHP = 128
TOT_CH = E_PAD // CHP
NRING = 2
NCH0 = 80
NCH1 = 80
assert 16 * (NCH0 + NCH1) == TOT_CH


@functools.partial(
    pl.kernel,
    out_type=jax.ShapeDtypeStruct((NC, N_PAD, D), _f32),
    mesh=_MESH,
    scratch_types=[
        pltpu.VMEM((NRING, CHP), _i32),
        pltpu.VMEM((NRING, CHP), _i32),
        pltpu.VMEM((NRING, CHP, D), _f32),
        pltpu.VMEM_SHARED((N_PAD, D), _f32),
        pltpu.SemaphoreType.DMA((NRING,)),
        pltpu.SemaphoreType.DMA((NRING,)),
        pltpu.SemaphoreType.DMA((NRING,)),
    ],
)
def _prop_kernel(xs_hbm, src_hbm, dst_hbm, parts_hbm,
                 isrc, idst, rows_v, agg, gsem, ssem, isem):
    cid = lax.axis_index("c")
    sid = lax.axis_index("s")
    base = jnp.where(cid == 1, sid * NCH1, NS * NCH1 + sid * NCH0)
    ngrp = jnp.where(cid == 1, NCH1 // NRING, NCH0 // NRING)

    def _z(r, _):
        for k in range(D // 16):
            rows_v[0, r, pl.ds(k * 16, 16)] = jnp.zeros((16,), _f32)
        return 0
    lax.fori_loop(0, CHP, _z, 0)
    for k in range(RPS // CHP):
        pltpu.sync_copy(rows_v.at[0], agg.at[pl.ds(sid * RPS + k * CHP, CHP)])
    plsc.subcore_barrier()

    def _idx_issue(j, k):
        pltpu.async_copy(src_hbm.at[base + j], isrc.at[k], isem.at[k])
        pltpu.async_copy(dst_hbm.at[base + j], idst.at[k], isem.at[k])

    def _idx_wait(j, k):
        pltpu.make_async_copy(src_hbm.at[base + j], isrc.at[k],
                              isem.at[k]).wait()
        pltpu.make_async_copy(dst_hbm.at[base + j], idst.at[k],
                              isem.at[k]).wait()

    def _gather(k):
        pltpu.async_copy(xs_hbm.at[isrc.at[k]], rows_v.at[k], gsem.at[k])

    def _gather_wait(k):
        pltpu.make_async_copy(xs_hbm.at[isrc.at[k]], rows_v.at[k],
                              gsem.at[k]).wait()

    def _scatter(k):
        pltpu.async_copy(rows_v.at[k], agg.at[idst.at[k]], ssem.at[k],
                         add=True)

    def _scatter_wait(k):
        pltpu.make_async_copy(rows_v.at[k], agg.at[idst.at[k]],
                              ssem.at[k]).wait()

    for k in range(NRING):
        _idx_issue(k, k)
    for k in range(NRING):
        _idx_wait(k, k)
        _gather(k)

    def _body(g, _):
        j0 = g * NRING
        for k in range(NRING):
            _gather_wait(k)
            _scatter(k)
        for k in range(NRING):
            @pl.when(g < ngrp - 1)
            def _():
                _scatter_wait(k)
                _idx_issue(j0 + NRING + k, k)
        for k in range(NRING):
            @pl.when(g < ngrp - 1)
            def _():
                _idx_wait(j0 + NRING + k, k)
                _gather(k)
        return 0
    lax.fori_loop(0, ngrp, _body, 0)
    for k in range(NRING):
        _scatter_wait(k)

    plsc.subcore_barrier()
    pltpu.sync_copy(agg.at[pl.ds(sid * RPS, RPS)],
                    parts_hbm.at[cid, pl.ds(sid * RPS, RPS)])


def _prep_body(do0, do1, di0, di1, feat, xs_out, nsrc_out, ndst_out):
    deg_o = do0[...] + do1[...]
    deg_i = di0[...] + di1[...]
    nsrc = jnp.where(deg_o > 0.0, 1.0 / jnp.sqrt(jnp.maximum(deg_o, 1.0)), 0.0)
    ndst = jnp.where(deg_i > 0.0, 1.0 / jnp.sqrt(jnp.maximum(deg_i, 1.0)), 0.0)
    nsrc_out[...] = nsrc
    ndst_out[...] = ndst
    xs_out[...] = feat[...] * nsrc


_ROWS_BLK = 2048
_N_BLKS = N_PAD // _ROWS_BLK


def _prep_call(do0, do1, di0, di1, feat):
    vec_spec = pl.BlockSpec((_ROWS_BLK, 1), lambda i: (i, 0))
    mat_spec = pl.BlockSpec((_ROWS_BLK, D), lambda i: (i, 0))
    return pl.pallas_call(
        _prep_body,
        grid=(_N_BLKS,),
        in_specs=[vec_spec, vec_spec, vec_spec, vec_spec, mat_spec],
        out_specs=[mat_spec, vec_spec, vec_spec],
        out_shape=[
            jax.ShapeDtypeStruct((N_PAD, D), _f32),
            jax.ShapeDtypeStruct((N_PAD, 1), _f32),
            jax.ShapeDtypeStruct((N_PAD, 1), _f32),
        ],
    )(do0, do1, di0, di1, feat)


def _layer_body(p0, p1, ndst, nsrc, w, b, h_out, xs_out):
    agg = (p0[...] + p1[...]) * ndst[...]
    h = jnp.dot(agg, w[...], preferred_element_type=_f32) + b[...]
    h = jnp.maximum(h, 0.0)
    h_out[...] = h
    xs_out[...] = h * nsrc[...]


def _layer_call(p0, p1, ndst, nsrc, w, b):
    vec_spec = pl.BlockSpec((_ROWS_BLK, 1), lambda i: (i, 0))
    mat_spec = pl.BlockSpec((_ROWS_BLK, D), lambda i: (i, 0))
    w_spec = pl.BlockSpec((D, D), lambda i: (0, 0))
    b_spec = pl.BlockSpec((1, D), lambda i: (0, 0))
    return pl.pallas_call(
        _layer_body,
        grid=(_N_BLKS,),
        in_specs=[mat_spec, mat_spec, vec_spec, vec_spec, w_spec, b_spec],
        out_specs=[mat_spec, mat_spec],
        out_shape=[
            jax.ShapeDtypeStruct((N_PAD, D), _f32),
            jax.ShapeDtypeStruct((N_PAD, D), _f32),
        ],
    )(p0, p1, ndst, nsrc, w, b)


def kernel(features, edge_index, W1, b1, W2, b2):
    src = edge_index[0]
    dst = edge_index[1]
    pad = PAD + jnp.arange(E_PAD - E, dtype=_i32) % (N_PAD - PAD)
    src_flat = jnp.concatenate([src, pad])
    dst_flat = jnp.concatenate([dst, pad])
    src_p = src_flat.reshape(NW, NCH, CH)
    dst_p = dst_flat.reshape(NW, NCH, CH)
    src_q = src_flat.reshape(TOT_CH, CHP)
    dst_q = dst_flat.reshape(TOT_CH, CHP)
    feat_pad = jnp.pad(features, ((0, N_PAD - N), (0, 0)))

    dout_p, din_p = _deg_kernel(src_p, dst_p)
    do0 = dout_p[0].reshape(N_PAD, 1)
    do1 = dout_p[1].reshape(N_PAD, 1)
    di0 = din_p[0].reshape(N_PAD, 1)
    di1 = din_p[1].reshape(N_PAD, 1)

    xs1, nsrc, ndst = _prep_call(do0, do1, di0, di1, feat_pad)

    b1r = b1.reshape(1, D)
    b2r = b2.reshape(1, D)

    parts1 = _prop_kernel(xs1, src_q, dst_q)
    _, xs2 = _layer_call(parts1[0], parts1[1], ndst, nsrc, W1, b1r)

    parts2 = _prop_kernel(xs2, src_q, dst_q)
    h2, _ = _layer_call(parts2[0], parts2[1], ndst, nsrc, W2, b2r)

    return h2[:N]


# speedup vs baseline: 1.1842x; 1.1842x over previous
import functools

import jax
import jax.numpy as jnp
from jax import lax
from jax.experimental import pallas as pl
from jax.experimental.pallas import tpu as pltpu
from jax.experimental.pallas import tpu_sc as plsc

N = 10000
E = 320000
D = 128

NC = 2
NS = 16
NW = NC * NS
CH = 128
NCH = 80
EPW = NCH * CH
E_PAD = NW * EPW
PAD = N
N_PAD = 10240
RPS = N_PAD // NS

_f32 = jnp.float32
_i32 = jnp.int32

_MESH = plsc.VectorSubcoreMesh(core_axis_name="c", subcore_axis_name="s")


@functools.partial(
    pl.kernel,
    out_type=(
        jax.ShapeDtypeStruct((NC, N_PAD), _f32),
        jax.ShapeDtypeStruct((NC, N_PAD), _f32),
    ),
    mesh=_MESH,
    scratch_types=[
        pltpu.VMEM((NCH, CH), _i32),
        pltpu.VMEM((NCH, CH), _i32),
        pltpu.VMEM((CH,), _f32),
        pltpu.VMEM((RPS,), _f32),
        pltpu.VMEM_SHARED((N_PAD,), _f32),
        pltpu.VMEM_SHARED((N_PAD,), _f32),
    ],
)
def _deg_kernel(src_hbm, dst_hbm, dout_hbm, din_hbm,
                src_v, dst_v, ones_v, zeros_v, deg_o, deg_i):
    cid = lax.axis_index("c")
    sid = lax.axis_index("s")
    wid = cid * NS + sid

    def _init(i, _):
        ones_v[pl.ds(i * 16, 16)] = jnp.ones((16,), _f32)
        return 0
    lax.fori_loop(0, CH // 16, _init, 0)

    def _zinit(i, _):
        zeros_v[pl.ds(i * 16, 16)] = jnp.zeros((16,), _f32)
        return 0
    lax.fori_loop(0, RPS // 16, _zinit, 0)

    pltpu.sync_copy(zeros_v, deg_o.at[pl.ds(sid * RPS, RPS)])
    pltpu.sync_copy(zeros_v, deg_i.at[pl.ds(sid * RPS, RPS)])
    plsc.subcore_barrier()

    pltpu.sync_copy(src_hbm.at[wid], src_v)
    pltpu.sync_copy(dst_hbm.at[wid], dst_v)

    def _body(j, _):
        pltpu.sync_copy(ones_v, deg_o.at[src_v.at[j]], add=True)
        pltpu.sync_copy(ones_v, deg_i.at[dst_v.at[j]], add=True)
        return 0
    lax.fori_loop(0, NCH, _body, 0)

    plsc.subcore_barrier()
    pltpu.sync_copy(deg_o.at[pl.ds(sid * RPS, RPS)],
                    dout_hbm.at[cid, pl.ds(sid * RPS, RPS)])
    pltpu.sync_copy(deg_i.at[pl.ds(sid * RPS, RPS)],
                    din_hbm.at[cid, pl.ds(sid * RPS, RPS)])


CHP = 32
TOT_CH = E_PAD // CHP
NRING = 10
NCH0 = 320
NCH1 = 320
assert 16 * (NCH0 + NCH1) == TOT_CH


@functools.partial(
    pl.kernel,
    out_type=jax.ShapeDtypeStruct((NC, N_PAD, D), _f32),
    mesh=_MESH,
    scratch_types=[
        pltpu.VMEM((NRING, CHP), _i32),
        pltpu.VMEM((NRING, CHP), _i32),
        pltpu.VMEM((NRING, CHP, D), _f32),
        pltpu.VMEM_SHARED((N_PAD, D), _f32),
        pltpu.SemaphoreType.DMA((NRING,)),
        pltpu.SemaphoreType.DMA((NRING,)),
        pltpu.SemaphoreType.DMA((NRING,)),
    ],
)
def _prop_kernel(xs_hbm, src_hbm, dst_hbm, parts_hbm,
                 isrc, idst, rows_v, agg, gsem, ssem, isem):
    cid = lax.axis_index("c")
    sid = lax.axis_index("s")
    base = jnp.where(cid == 1, sid * NCH1, NS * NCH1 + sid * NCH0)
    ngrp = jnp.where(cid == 1, NCH1 // NRING, NCH0 // NRING)

    def _z(r, _):
        for k in range(D // 16):
            rows_v[0, r, pl.ds(k * 16, 16)] = jnp.zeros((16,), _f32)
        return 0
    lax.fori_loop(0, CHP, _z, 0)
    for k in range(RPS // CHP):
        pltpu.sync_copy(rows_v.at[0], agg.at[pl.ds(sid * RPS + k * CHP, CHP)])
    plsc.subcore_barrier()

    def _idx_issue(j, k):
        pltpu.async_copy(src_hbm.at[base + j], isrc.at[k], isem.at[k])
        pltpu.async_copy(dst_hbm.at[base + j], idst.at[k], isem.at[k])

    def _idx_wait(j, k):
        pltpu.make_async_copy(src_hbm.at[base + j], isrc.at[k],
                              isem.at[k]).wait()
        pltpu.make_async_copy(dst_hbm.at[base + j], idst.at[k],
                              isem.at[k]).wait()

    def _gather(k):
        pltpu.async_copy(xs_hbm.at[isrc.at[k]], rows_v.at[k], gsem.at[k])

    def _gather_wait(k):
        pltpu.make_async_copy(xs_hbm.at[isrc.at[k]], rows_v.at[k],
                              gsem.at[k]).wait()

    def _scatter(k):
        pltpu.async_copy(rows_v.at[k], agg.at[idst.at[k]], ssem.at[k],
                         add=True)

    def _scatter_wait(k):
        pltpu.make_async_copy(rows_v.at[k], agg.at[idst.at[k]],
                              ssem.at[k]).wait()

    for k in range(NRING):
        _idx_issue(k, k)
    for k in range(NRING):
        _idx_wait(k, k)
        _gather(k)

    def _body(g, _):
        j0 = g * NRING
        for k in range(NRING):
            _gather_wait(k)
            _scatter(k)
        for k in range(NRING):
            @pl.when(g < ngrp - 1)
            def _():
                _scatter_wait(k)
                _idx_issue(j0 + NRING + k, k)
        for k in range(NRING):
            @pl.when(g < ngrp - 1)
            def _():
                _idx_wait(j0 + NRING + k, k)
                _gather(k)
        return 0
    lax.fori_loop(0, ngrp, _body, 0)
    for k in range(NRING):
        _scatter_wait(k)

    plsc.subcore_barrier()
    pltpu.sync_copy(agg.at[pl.ds(sid * RPS, RPS)],
                    parts_hbm.at[cid, pl.ds(sid * RPS, RPS)])


def _prep_body(do0, do1, di0, di1, feat, xs_out, nsrc_out, ndst_out):
    deg_o = do0[...] + do1[...]
    deg_i = di0[...] + di1[...]
    nsrc = jnp.where(deg_o > 0.0, 1.0 / jnp.sqrt(jnp.maximum(deg_o, 1.0)), 0.0)
    ndst = jnp.where(deg_i > 0.0, 1.0 / jnp.sqrt(jnp.maximum(deg_i, 1.0)), 0.0)
    nsrc_out[...] = nsrc
    ndst_out[...] = ndst
    xs_out[...] = feat[...] * nsrc


_ROWS_BLK = 2048
_N_BLKS = N_PAD // _ROWS_BLK


def _prep_call(do0, do1, di0, di1, feat):
    vec_spec = pl.BlockSpec((_ROWS_BLK, 1), lambda i: (i, 0))
    mat_spec = pl.BlockSpec((_ROWS_BLK, D), lambda i: (i, 0))
    return pl.pallas_call(
        _prep_body,
        grid=(_N_BLKS,),
        in_specs=[vec_spec, vec_spec, vec_spec, vec_spec, mat_spec],
        out_specs=[mat_spec, vec_spec, vec_spec],
        out_shape=[
            jax.ShapeDtypeStruct((N_PAD, D), _f32),
            jax.ShapeDtypeStruct((N_PAD, 1), _f32),
            jax.ShapeDtypeStruct((N_PAD, 1), _f32),
        ],
    )(do0, do1, di0, di1, feat)


def _layer_body(p0, p1, ndst, nsrc, w, b, h_out, xs_out):
    agg = (p0[...] + p1[...]) * ndst[...]
    h = jnp.dot(agg, w[...], preferred_element_type=_f32) + b[...]
    h = jnp.maximum(h, 0.0)
    h_out[...] = h
    xs_out[...] = h * nsrc[...]


def _layer_call(p0, p1, ndst, nsrc, w, b):
    vec_spec = pl.BlockSpec((_ROWS_BLK, 1), lambda i: (i, 0))
    mat_spec = pl.BlockSpec((_ROWS_BLK, D), lambda i: (i, 0))
    w_spec = pl.BlockSpec((D, D), lambda i: (0, 0))
    b_spec = pl.BlockSpec((1, D), lambda i: (0, 0))
    return pl.pallas_call(
        _layer_body,
        grid=(_N_BLKS,),
        in_specs=[mat_spec, mat_spec, vec_spec, vec_spec, w_spec, b_spec],
        out_specs=[mat_spec, mat_spec],
        out_shape=[
            jax.ShapeDtypeStruct((N_PAD, D), _f32),
            jax.ShapeDtypeStruct((N_PAD, D), _f32),
        ],
    )(p0, p1, ndst, nsrc, w, b)


def kernel(features, edge_index, W1, b1, W2, b2):
    src = edge_index[0]
    dst = edge_index[1]
    pad = PAD + jnp.arange(E_PAD - E, dtype=_i32) % (N_PAD - PAD)
    src_flat = jnp.concatenate([src, pad])
    dst_flat = jnp.concatenate([dst, pad])
    src_p = src_flat.reshape(NW, NCH, CH)
    dst_p = dst_flat.reshape(NW, NCH, CH)
    src_q = src_flat.reshape(TOT_CH, CHP)
    dst_q = dst_flat.reshape(TOT_CH, CHP)
    feat_pad = jnp.pad(features, ((0, N_PAD - N), (0, 0)))

    dout_p, din_p = _deg_kernel(src_p, dst_p)
    do0 = dout_p[0].reshape(N_PAD, 1)
    do1 = dout_p[1].reshape(N_PAD, 1)
    di0 = din_p[0].reshape(N_PAD, 1)
    di1 = din_p[1].reshape(N_PAD, 1)

    xs1, nsrc, ndst = _prep_call(do0, do1, di0, di1, feat_pad)

    b1r = b1.reshape(1, D)
    b2r = b2.reshape(1, D)

    parts1 = _prop_kernel(xs1, src_q, dst_q)
    _, xs2 = _layer_call(parts1[0], parts1[1], ndst, nsrc, W1, b1r)

    parts2 = _prop_kernel(xs2, src_q, dst_q)
    h2, _ = _layer_call(parts2[0], parts2[1], ndst, nsrc, W2, b2r)

    return h2[:N]


# speedup vs baseline: 1.2620x; 1.0657x over previous
import functools

import jax
import jax.numpy as jnp
from jax import lax
from jax.experimental import pallas as pl
from jax.experimental.pallas import tpu as pltpu
from jax.experimental.pallas import tpu_sc as plsc

N = 10000
E = 320000
D = 128

NC = 2
NS = 16
NW = NC * NS
CH = 128
NCH = 80
EPW = NCH * CH
E_PAD = NW * EPW
PAD = N
N_PAD = 10240
RPS = N_PAD // NS

_f32 = jnp.float32
_i32 = jnp.int32

_MESH = plsc.VectorSubcoreMesh(core_axis_name="c", subcore_axis_name="s")


@functools.partial(
    pl.kernel,
    out_type=(
        jax.ShapeDtypeStruct((NC, N_PAD), _f32),
        jax.ShapeDtypeStruct((NC, N_PAD), _f32),
    ),
    mesh=_MESH,
    scratch_types=[
        pltpu.VMEM((NCH, CH), _i32),
        pltpu.VMEM((NCH, CH), _i32),
        pltpu.VMEM((CH,), _f32),
        pltpu.VMEM((RPS,), _f32),
        pltpu.VMEM_SHARED((N_PAD,), _f32),
        pltpu.VMEM_SHARED((N_PAD,), _f32),
    ],
)
def _deg_kernel(src_hbm, dst_hbm, dout_hbm, din_hbm,
                src_v, dst_v, ones_v, zeros_v, deg_o, deg_i):
    cid = lax.axis_index("c")
    sid = lax.axis_index("s")
    wid = cid * NS + sid

    def _init(i, _):
        ones_v[pl.ds(i * 16, 16)] = jnp.ones((16,), _f32)
        return 0
    lax.fori_loop(0, CH // 16, _init, 0)

    def _zinit(i, _):
        zeros_v[pl.ds(i * 16, 16)] = jnp.zeros((16,), _f32)
        return 0
    lax.fori_loop(0, RPS // 16, _zinit, 0)

    pltpu.sync_copy(zeros_v, deg_o.at[pl.ds(sid * RPS, RPS)])
    pltpu.sync_copy(zeros_v, deg_i.at[pl.ds(sid * RPS, RPS)])
    plsc.subcore_barrier()

    pltpu.sync_copy(src_hbm.at[wid], src_v)
    pltpu.sync_copy(dst_hbm.at[wid], dst_v)

    def _body(j, _):
        pltpu.sync_copy(ones_v, deg_o.at[src_v.at[j]], add=True)
        pltpu.sync_copy(ones_v, deg_i.at[dst_v.at[j]], add=True)
        return 0
    lax.fori_loop(0, NCH, _body, 0)

    plsc.subcore_barrier()
    pltpu.sync_copy(deg_o.at[pl.ds(sid * RPS, RPS)],
                    dout_hbm.at[cid, pl.ds(sid * RPS, RPS)])
    pltpu.sync_copy(deg_i.at[pl.ds(sid * RPS, RPS)],
                    din_hbm.at[cid, pl.ds(sid * RPS, RPS)])


CHP = 32
TOT_CH = E_PAD // CHP
NRING = 10
NCH0 = 320
NCH1 = 320
assert 16 * (NCH0 + NCH1) == TOT_CH


@functools.partial(
    pl.kernel,
    out_type=jax.ShapeDtypeStruct((NC, N_PAD, D), _f32),
    mesh=_MESH,
    scratch_types=[
        pltpu.VMEM((NRING, CHP), _i32),
        pltpu.VMEM((NRING, CHP), _i32),
        pltpu.VMEM((NRING, CHP, D), _f32),
        pltpu.VMEM_SHARED((N_PAD, D), _f32),
        pltpu.SemaphoreType.DMA((NRING,)),
        pltpu.SemaphoreType.DMA((NRING,)),
        pltpu.SemaphoreType.DMA((NRING,)),
    ],
)
def _prop_kernel(xs_hbm, src_hbm, dst_hbm, parts_hbm,
                 isrc, idst, rows_v, agg, gsem, ssem, isem):
    cid = lax.axis_index("c")
    sid = lax.axis_index("s")
    base = jnp.where(cid == 1, sid * NCH1, NS * NCH1 + sid * NCH0)
    ngrp = jnp.where(cid == 1, NCH1 // NRING, NCH0 // NRING)

    def _z(r, _):
        for k in range(D // 16):
            rows_v[0, r, pl.ds(k * 16, 16)] = jnp.zeros((16,), _f32)
        return 0
    lax.fori_loop(0, CHP, _z, 0)
    for k in range(RPS // CHP):
        pltpu.sync_copy(rows_v.at[0], agg.at[pl.ds(sid * RPS + k * CHP, CHP)])
    plsc.subcore_barrier()

    def _idx_issue(j, k):
        pltpu.async_copy(src_hbm.at[base + j], isrc.at[k], isem.at[k])
        pltpu.async_copy(dst_hbm.at[base + j], idst.at[k], isem.at[k])

    def _idx_wait(j, k):
        pltpu.make_async_copy(src_hbm.at[base + j], isrc.at[k],
                              isem.at[k]).wait()
        pltpu.make_async_copy(dst_hbm.at[base + j], idst.at[k],
                              isem.at[k]).wait()

    def _gather(k):
        pltpu.async_copy(xs_hbm.at[isrc.at[k]], rows_v.at[k], gsem.at[k])

    def _gather_wait(k):
        pltpu.make_async_copy(xs_hbm.at[isrc.at[k]], rows_v.at[k],
                              gsem.at[k]).wait()

    def _scatter(k):
        pltpu.async_copy(rows_v.at[k], agg.at[idst.at[k]], ssem.at[k],
                         add=True)

    def _scatter_wait(k):
        pltpu.make_async_copy(rows_v.at[k], agg.at[idst.at[k]],
                              ssem.at[k]).wait()

    for k in range(NRING):
        _idx_issue(k, k)
    for k in range(NRING):
        _idx_wait(k, k)
        _gather(k)

    def _body(g, _):
        j0 = g * NRING
        for k in range(NRING):
            _gather_wait(k)
            _scatter(k)
        for k in range(NRING):
            @pl.when(g < ngrp - 1)
            def _():
                _scatter_wait(k)
                _idx_issue(j0 + NRING + k, k)
        for k in range(NRING):
            @pl.when(g < ngrp - 1)
            def _():
                _idx_wait(j0 + NRING + k, k)
                _gather(k)
        return 0
    lax.fori_loop(0, ngrp, _body, 0)
    for k in range(NRING):
        _scatter_wait(k)

    plsc.subcore_barrier()
    pltpu.sync_copy(agg.at[pl.ds(sid * RPS, RPS)],
                    parts_hbm.at[cid, pl.ds(sid * RPS, RPS)])


def _prep_body(do0, do1, di0, di1, feat, xs_out, nsrc_out, ndst_out):
    deg_o = do0[...] + do1[...]
    deg_i = di0[...] + di1[...]
    nsrc = jnp.where(deg_o > 0.0, 1.0 / jnp.sqrt(jnp.maximum(deg_o, 1.0)), 0.0)
    ndst = jnp.where(deg_i > 0.0, 1.0 / jnp.sqrt(jnp.maximum(deg_i, 1.0)), 0.0)
    nsrc_out[...] = nsrc
    ndst_out[...] = ndst
    xs_out[...] = feat[...] * nsrc


_ROWS_BLK = 2048
_N_BLKS = N_PAD // _ROWS_BLK


def _prep_call(do0, do1, di0, di1, feat):
    vec_spec = pl.BlockSpec((_ROWS_BLK, 1), lambda i: (i, 0))
    mat_spec = pl.BlockSpec((_ROWS_BLK, D), lambda i: (i, 0))
    return pl.pallas_call(
        _prep_body,
        grid=(_N_BLKS,),
        in_specs=[vec_spec, vec_spec, vec_spec, vec_spec, mat_spec],
        out_specs=[mat_spec, vec_spec, vec_spec],
        out_shape=[
            jax.ShapeDtypeStruct((N_PAD, D), _f32),
            jax.ShapeDtypeStruct((N_PAD, 1), _f32),
            jax.ShapeDtypeStruct((N_PAD, 1), _f32),
        ],
    )(do0, do1, di0, di1, feat)


def _mid_layer_body(p0, p1, ndst, nsrc, w, b, xs_out):
    agg = (p0[0] + p1[0]) * ndst[...]
    h = jnp.dot(agg, w[...], preferred_element_type=_f32) + b[...]
    xs_out[...] = jnp.maximum(h, 0.0) * nsrc[...]


def _mid_layer_call(parts, ndst, nsrc, w, b):
    vec_spec = pl.BlockSpec((_ROWS_BLK, 1), lambda i: (i, 0))
    mat_spec = pl.BlockSpec((_ROWS_BLK, D), lambda i: (i, 0))
    p0_spec = pl.BlockSpec((1, _ROWS_BLK, D), lambda i: (0, i, 0))
    p1_spec = pl.BlockSpec((1, _ROWS_BLK, D), lambda i: (1, i, 0))
    w_spec = pl.BlockSpec((D, D), lambda i: (0, 0))
    b_spec = pl.BlockSpec((1, D), lambda i: (0, 0))
    return pl.pallas_call(
        _mid_layer_body,
        grid=(_N_BLKS,),
        in_specs=[p0_spec, p1_spec, vec_spec, vec_spec, w_spec, b_spec],
        out_specs=mat_spec,
        out_shape=jax.ShapeDtypeStruct((N_PAD, D), _f32),
    )(parts, parts, ndst, nsrc, w, b)


_FIN_BLK = 2000


def _fin_layer_body(p0, p1, ndst, w, b, h_out):
    agg = (p0[0] + p1[0]) * ndst[...]
    h = jnp.dot(agg, w[...], preferred_element_type=_f32) + b[...]
    h_out[...] = jnp.maximum(h, 0.0)


def _fin_layer_call(parts, ndst, w, b):
    vec_spec = pl.BlockSpec((_FIN_BLK, 1), lambda i: (i, 0))
    mat_spec = pl.BlockSpec((_FIN_BLK, D), lambda i: (i, 0))
    p0_spec = pl.BlockSpec((1, _FIN_BLK, D), lambda i: (0, i, 0))
    p1_spec = pl.BlockSpec((1, _FIN_BLK, D), lambda i: (1, i, 0))
    w_spec = pl.BlockSpec((D, D), lambda i: (0, 0))
    b_spec = pl.BlockSpec((1, D), lambda i: (0, 0))
    return pl.pallas_call(
        _fin_layer_body,
        grid=(N // _FIN_BLK,),
        in_specs=[p0_spec, p1_spec, vec_spec, w_spec, b_spec],
        out_specs=mat_spec,
        out_shape=jax.ShapeDtypeStruct((N, D), _f32),
    )(parts, parts, ndst, w, b)


def kernel(features, edge_index, W1, b1, W2, b2):
    src = edge_index[0]
    dst = edge_index[1]
    pad = PAD + jnp.arange(E_PAD - E, dtype=_i32) % (N_PAD - PAD)
    src_flat = jnp.concatenate([src, pad])
    dst_flat = jnp.concatenate([dst, pad])
    src_p = src_flat.reshape(NW, NCH, CH)
    dst_p = dst_flat.reshape(NW, NCH, CH)
    src_q = src_flat.reshape(TOT_CH, CHP)
    dst_q = dst_flat.reshape(TOT_CH, CHP)
    feat_pad = jnp.pad(features, ((0, N_PAD - N), (0, 0)))

    dout_p, din_p = _deg_kernel(src_p, dst_p)
    do0 = dout_p[0].reshape(N_PAD, 1)
    do1 = dout_p[1].reshape(N_PAD, 1)
    di0 = din_p[0].reshape(N_PAD, 1)
    di1 = din_p[1].reshape(N_PAD, 1)

    xs1, nsrc, ndst = _prep_call(do0, do1, di0, di1, feat_pad)

    b1r = b1.reshape(1, D)
    b2r = b2.reshape(1, D)

    parts1 = _prop_kernel(xs1, src_q, dst_q)
    xs2 = _mid_layer_call(parts1, ndst, nsrc, W1, b1r)

    parts2 = _prop_kernel(xs2, src_q, dst_q)
    return _fin_layer_call(parts2, ndst, W2, b2r)


# speedup vs baseline: 1.3016x; 1.0314x over previous
import functools

import jax
import jax.numpy as jnp
from jax import lax
from jax.experimental import pallas as pl
from jax.experimental.pallas import tpu as pltpu
from jax.experimental.pallas import tpu_sc as plsc

N = 10000
E = 320000
D = 128

NC = 2
NS = 16
NW = NC * NS
CH = 128
NCH = 80
EPW = NCH * CH
E_PAD = NW * EPW
PAD = N
N_PAD = 10240
RPS = N_PAD // NS

_f32 = jnp.float32
_i32 = jnp.int32

_MESH = plsc.VectorSubcoreMesh(core_axis_name="c", subcore_axis_name="s")


@functools.partial(
    pl.kernel,
    out_type=(
        jax.ShapeDtypeStruct((NC, N_PAD), _f32),
        jax.ShapeDtypeStruct((NC, N_PAD), _f32),
    ),
    mesh=_MESH,
    scratch_types=[
        pltpu.VMEM((NCH, CH), _i32),
        pltpu.VMEM((NCH, CH), _i32),
        pltpu.VMEM((CH,), _f32),
        pltpu.VMEM((RPS,), _f32),
        pltpu.VMEM_SHARED((N_PAD,), _f32),
        pltpu.VMEM_SHARED((N_PAD,), _f32),
        pltpu.SemaphoreType.DMA((8,)),
    ],
)
def _deg_kernel(src_hbm, dst_hbm, dout_hbm, din_hbm,
                src_v, dst_v, ones_v, zeros_v, deg_o, deg_i, dsem):
    cid = lax.axis_index("c")
    sid = lax.axis_index("s")
    wid = cid * NS + sid

    def _init(i, _):
        ones_v[pl.ds(i * 16, 16)] = jnp.ones((16,), _f32)
        return 0
    lax.fori_loop(0, CH // 16, _init, 0)

    def _zinit(i, _):
        zeros_v[pl.ds(i * 16, 16)] = jnp.zeros((16,), _f32)
        return 0
    lax.fori_loop(0, RPS // 16, _zinit, 0)

    pltpu.sync_copy(zeros_v, deg_o.at[pl.ds(sid * RPS, RPS)])
    pltpu.sync_copy(zeros_v, deg_i.at[pl.ds(sid * RPS, RPS)])
    plsc.subcore_barrier()

    pltpu.sync_copy(src_hbm.at[wid], src_v)
    pltpu.sync_copy(dst_hbm.at[wid], dst_v)

    def _dsc(j, k, ref, idx):
        pltpu.async_copy(ones_v, ref.at[idx.at[j]], dsem.at[k], add=True)

    def _dsc_wait(j, k, ref, idx):
        pltpu.make_async_copy(ones_v, ref.at[idx.at[j]], dsem.at[k]).wait()

    for k in range(4):
        _dsc(k, k, deg_o, src_v)
        _dsc(k, k + 4, deg_i, dst_v)

    def _body(g, _):
        j0 = g * 4
        for k in range(4):
            @pl.when(g < NCH // 4 - 1)
            def _():
                _dsc_wait(j0 + k, k, deg_o, src_v)
                _dsc(j0 + 4 + k, k, deg_o, src_v)
                _dsc_wait(j0 + k, k + 4, deg_i, dst_v)
                _dsc(j0 + 4 + k, k + 4, deg_i, dst_v)
        return 0
    lax.fori_loop(0, NCH // 4, _body, 0)
    for k in range(4):
        _dsc_wait(NCH - 4 + k, k, deg_o, src_v)
        _dsc_wait(NCH - 4 + k, k + 4, deg_i, dst_v)

    plsc.subcore_barrier()
    pltpu.sync_copy(deg_o.at[pl.ds(sid * RPS, RPS)],
                    dout_hbm.at[cid, pl.ds(sid * RPS, RPS)])
    pltpu.sync_copy(deg_i.at[pl.ds(sid * RPS, RPS)],
                    din_hbm.at[cid, pl.ds(sid * RPS, RPS)])


CHP = 32
TOT_CH = E_PAD // CHP
NRING = 10
NCH0 = 320
NCH1 = 320
assert 16 * (NCH0 + NCH1) == TOT_CH


@functools.partial(
    pl.kernel,
    out_type=jax.ShapeDtypeStruct((NC, N_PAD, D), _f32),
    mesh=_MESH,
    scratch_types=[
        pltpu.VMEM((NRING, CHP), _i32),
        pltpu.VMEM((NRING, CHP), _i32),
        pltpu.VMEM((NRING, CHP, D), _f32),
        pltpu.VMEM_SHARED((N_PAD, D), _f32),
        pltpu.SemaphoreType.DMA((NRING,)),
        pltpu.SemaphoreType.DMA((NRING,)),
        pltpu.SemaphoreType.DMA((NRING,)),
    ],
)
def _prop_kernel(xs_hbm, src_hbm, dst_hbm, parts_hbm,
                 isrc, idst, rows_v, agg, gsem, ssem, isem):
    cid = lax.axis_index("c")
    sid = lax.axis_index("s")
    base = jnp.where(cid == 1, sid * NCH1, NS * NCH1 + sid * NCH0)
    ngrp = jnp.where(cid == 1, NCH1 // NRING, NCH0 // NRING)

    def _z(r, _):
        for k in range(D // 16):
            rows_v[0, r, pl.ds(k * 16, 16)] = jnp.zeros((16,), _f32)
        return 0
    lax.fori_loop(0, CHP, _z, 0)
    for k in range(RPS // CHP):
        pltpu.sync_copy(rows_v.at[0], agg.at[pl.ds(sid * RPS + k * CHP, CHP)])
    plsc.subcore_barrier()

    def _idx_issue(j, k):
        pltpu.async_copy(src_hbm.at[base + j], isrc.at[k], isem.at[k])
        pltpu.async_copy(dst_hbm.at[base + j], idst.at[k], isem.at[k])

    def _idx_wait(j, k):
        pltpu.make_async_copy(src_hbm.at[base + j], isrc.at[k],
                              isem.at[k]).wait()
        pltpu.make_async_copy(dst_hbm.at[base + j], idst.at[k],
                              isem.at[k]).wait()

    def _gather(k):
        pltpu.async_copy(xs_hbm.at[isrc.at[k]], rows_v.at[k], gsem.at[k])

    def _gather_wait(k):
        pltpu.make_async_copy(xs_hbm.at[isrc.at[k]], rows_v.at[k],
                              gsem.at[k]).wait()

    def _scatter(k):
        pltpu.async_copy(rows_v.at[k], agg.at[idst.at[k]], ssem.at[k],
                         add=True)

    def _scatter_wait(k):
        pltpu.make_async_copy(rows_v.at[k], agg.at[idst.at[k]],
                              ssem.at[k]).wait()

    for k in range(NRING):
        _idx_issue(k, k)
    for k in range(NRING):
        _idx_wait(k, k)
        _gather(k)

    def _body(g, _):
        j0 = g * NRING
        for k in range(NRING):
            _gather_wait(k)
            _scatter(k)
        for k in range(NRING):
            @pl.when(g < ngrp - 1)
            def _():
                _scatter_wait(k)
                _idx_issue(j0 + NRING + k, k)
        for k in range(NRING):
            @pl.when(g < ngrp - 1)
            def _():
                _idx_wait(j0 + NRING + k, k)
                _gather(k)
        return 0
    lax.fori_loop(0, ngrp, _body, 0)
    for k in range(NRING):
        _scatter_wait(k)

    plsc.subcore_barrier()
    pltpu.sync_copy(agg.at[pl.ds(sid * RPS, RPS)],
                    parts_hbm.at[cid, pl.ds(sid * RPS, RPS)])


def _prep_body(do0, do1, di0, di1, feat, xs_out, nsrc_out, ndst_out):
    deg_o = do0[...] + do1[...]
    deg_i = di0[...] + di1[...]
    nsrc = jnp.where(deg_o > 0.0, 1.0 / jnp.sqrt(jnp.maximum(deg_o, 1.0)), 0.0)
    ndst = jnp.where(deg_i > 0.0, 1.0 / jnp.sqrt(jnp.maximum(deg_i, 1.0)), 0.0)
    nsrc_out[...] = nsrc
    ndst_out[...] = ndst
    xs_out[...] = feat[...] * nsrc


_ROWS_BLK = 2048
_N_BLKS = N_PAD // _ROWS_BLK


def _prep_call(do0, do1, di0, di1, feat):
    vec_spec = pl.BlockSpec((_ROWS_BLK, 1), lambda i: (i, 0))
    mat_spec = pl.BlockSpec((_ROWS_BLK, D), lambda i: (i, 0))
    return pl.pallas_call(
        _prep_body,
        grid=(_N_BLKS,),
        in_specs=[vec_spec, vec_spec, vec_spec, vec_spec, mat_spec],
        out_specs=[mat_spec, vec_spec, vec_spec],
        out_shape=[
            jax.ShapeDtypeStruct((N_PAD, D), _f32),
            jax.ShapeDtypeStruct((N_PAD, 1), _f32),
            jax.ShapeDtypeStruct((N_PAD, 1), _f32),
        ],
    )(do0, do1, di0, di1, feat)


def _mid_layer_body(p0, p1, ndst, nsrc, w, b, xs_out):
    agg = (p0[0] + p1[0]) * ndst[...]
    h = jnp.dot(agg, w[...], preferred_element_type=_f32) + b[...]
    xs_out[...] = jnp.maximum(h, 0.0) * nsrc[...]


def _mid_layer_call(parts, ndst, nsrc, w, b):
    vec_spec = pl.BlockSpec((_ROWS_BLK, 1), lambda i: (i, 0))
    mat_spec = pl.BlockSpec((_ROWS_BLK, D), lambda i: (i, 0))
    p0_spec = pl.BlockSpec((1, _ROWS_BLK, D), lambda i: (0, i, 0))
    p1_spec = pl.BlockSpec((1, _ROWS_BLK, D), lambda i: (1, i, 0))
    w_spec = pl.BlockSpec((D, D), lambda i: (0, 0))
    b_spec = pl.BlockSpec((1, D), lambda i: (0, 0))
    return pl.pallas_call(
        _mid_layer_body,
        grid=(_N_BLKS,),
        in_specs=[p0_spec, p1_spec, vec_spec, vec_spec, w_spec, b_spec],
        out_specs=mat_spec,
        out_shape=jax.ShapeDtypeStruct((N_PAD, D), _f32),
    )(parts, parts, ndst, nsrc, w, b)


_FIN_BLK = 2000


def _fin_layer_body(p0, p1, ndst, w, b, h_out):
    agg = (p0[0] + p1[0]) * ndst[...]
    h = jnp.dot(agg, w[...], preferred_element_type=_f32) + b[...]
    h_out[...] = jnp.maximum(h, 0.0)


def _fin_layer_call(parts, ndst, w, b):
    vec_spec = pl.BlockSpec((_FIN_BLK, 1), lambda i: (i, 0))
    mat_spec = pl.BlockSpec((_FIN_BLK, D), lambda i: (i, 0))
    p0_spec = pl.BlockSpec((1, _FIN_BLK, D), lambda i: (0, i, 0))
    p1_spec = pl.BlockSpec((1, _FIN_BLK, D), lambda i: (1, i, 0))
    w_spec = pl.BlockSpec((D, D), lambda i: (0, 0))
    b_spec = pl.BlockSpec((1, D), lambda i: (0, 0))
    return pl.pallas_call(
        _fin_layer_body,
        grid=(N // _FIN_BLK,),
        in_specs=[p0_spec, p1_spec, vec_spec, w_spec, b_spec],
        out_specs=mat_spec,
        out_shape=jax.ShapeDtypeStruct((N, D), _f32),
    )(parts, parts, ndst, w, b)


def kernel(features, edge_index, W1, b1, W2, b2):
    src = edge_index[0]
    dst = edge_index[1]
    pad = PAD + jnp.arange(E_PAD - E, dtype=_i32) % (N_PAD - PAD)
    src_flat = jnp.concatenate([src, pad])
    dst_flat = jnp.concatenate([dst, pad])
    src_p = src_flat.reshape(NW, NCH, CH)
    dst_p = dst_flat.reshape(NW, NCH, CH)
    src_q = src_flat.reshape(TOT_CH, CHP)
    dst_q = dst_flat.reshape(TOT_CH, CHP)
    feat_pad = jnp.pad(features, ((0, N_PAD - N), (0, 0)))

    dout_p, din_p = _deg_kernel(src_p, dst_p)
    do0 = dout_p[0].reshape(N_PAD, 1)
    do1 = dout_p[1].reshape(N_PAD, 1)
    di0 = din_p[0].reshape(N_PAD, 1)
    di1 = din_p[1].reshape(N_PAD, 1)

    xs1, nsrc, ndst = _prep_call(do0, do1, di0, di1, feat_pad)

    b1r = b1.reshape(1, D)
    b2r = b2.reshape(1, D)

    parts1 = _prop_kernel(xs1, src_q, dst_q)
    xs2 = _mid_layer_call(parts1, ndst, nsrc, W1, b1r)

    parts2 = _prop_kernel(xs2, src_q, dst_q)
    return _fin_layer_call(parts2, ndst, W2, b2r)
